# baseline (speedup 1.0000x reference)
import numpy as np
import jax
import jax.numpy as jnp
from jax.experimental import pallas as pl

_D_MODEL = 1024
_BLOCK = 2048


def _make_tables(seq_len, d_model, block):
    n_blocks = seq_len // block
    c = np.arange(d_model, dtype=np.float64)
    denom = np.power(10000.0, c / d_model)
    even = (np.arange(d_model) % 2 == 0)[None, :]

    alpha = (block * np.arange(n_blocks, dtype=np.float64))[:, None] / denom[None, :]
    a1 = np.where(even, np.sin(alpha), np.cos(alpha)).astype(np.float32)[:, None, :]
    a2 = np.where(even, np.cos(alpha), -np.sin(alpha)).astype(np.float32)[:, None, :]

    beta = np.arange(block, dtype=np.float64)[:, None] / denom[None, :]
    b1 = np.cos(beta).astype(np.float32)
    b2 = np.sin(beta).astype(np.float32)
    return a1, a2, b1, b2


def _gen_body(a1_ref, a2_ref, b1_ref, b2_ref, out_ref):
    out_ref[...] = a1_ref[0] * b1_ref[...] + a2_ref[0] * b2_ref[...]


def kernel(x, encoding):
    seq_len = x.shape[0]
    d_model = encoding.shape[1]
    block = _BLOCK
    n_blocks = seq_len // block
    a1, a2, b1, b2 = _make_tables(seq_len, d_model, block)
    return pl.pallas_call(
        _gen_body,
        grid=(n_blocks,),
        in_specs=[
            pl.BlockSpec((1, 1, d_model), lambda i: (i, 0, 0)),
            pl.BlockSpec((1, 1, d_model), lambda i: (i, 0, 0)),
            pl.BlockSpec((block, d_model), lambda i: (0, 0)),
            pl.BlockSpec((block, d_model), lambda i: (0, 0)),
        ],
        out_specs=pl.BlockSpec((block, d_model), lambda i: (i, 0)),
        out_shape=jax.ShapeDtypeStruct((seq_len, d_model), jnp.float32),
    )(a1, a2, b1, b2)


# speedup vs baseline: 1.4386x; 1.4386x over previous
import numpy as np
import jax
import jax.numpy as jnp
from jax.experimental import pallas as pl

_BLOCK = 1024
_FINE = 128


def _make_tables(seq_len, d_model, block, fine):
    n_blocks = seq_len // block
    n_mid = block // fine
    c = np.arange(d_model, dtype=np.float64)
    denom = np.power(10000.0, c / d_model)
    even = (np.arange(d_model) % 2 == 0)[None, :]

    alpha = (block * np.arange(n_blocks, dtype=np.float64))[:, None] / denom[None, :]
    a1 = np.where(even, np.sin(alpha), np.cos(alpha)).astype(np.float32)[:, None, :]
    a2 = np.where(even, np.cos(alpha), -np.sin(alpha)).astype(np.float32)[:, None, :]

    mu = (fine * np.arange(n_mid, dtype=np.float64))[:, None] / denom[None, :]
    m1 = np.cos(mu).astype(np.float32)
    m2 = np.sin(mu).astype(np.float32)

    beta = np.arange(fine, dtype=np.float64)[:, None] / denom[None, :]
    b1 = np.cos(beta).astype(np.float32)
    b2 = np.sin(beta).astype(np.float32)
    return a1, a2, m1, m2, b1, b2


def _gen_body(a1_ref, a2_ref, m1_ref, m2_ref, b1_ref, b2_ref, out_ref):
    a1 = a1_ref[0]
    a2 = a2_ref[0]
    b1 = b1_ref[...]
    b2 = b2_ref[...]
    n_mid = m1_ref.shape[0]
    fine = b1.shape[0]
    for m in range(n_mid):
        m1 = m1_ref[m][None, :]
        m2 = m2_ref[m][None, :]
        g1 = a1 * m1 + a2 * m2
        g2 = a2 * m1 - a1 * m2
        out_ref[m * fine:(m + 1) * fine, :] = g1 * b1 + g2 * b2


def kernel(x, encoding):
    seq_len = x.shape[0]
    d_model = encoding.shape[1]
    block = _BLOCK
    fine = _FINE
    n_blocks = seq_len // block
    n_mid = block // fine
    a1, a2, m1, m2, b1, b2 = _make_tables(seq_len, d_model, block, fine)
    return pl.pallas_call(
        _gen_body,
        grid=(n_blocks,),
        in_specs=[
            pl.BlockSpec((1, 1, d_model), lambda i: (i, 0, 0)),
            pl.BlockSpec((1, 1, d_model), lambda i: (i, 0, 0)),
            pl.BlockSpec((n_mid, d_model), lambda i: (0, 0)),
            pl.BlockSpec((n_mid, d_model), lambda i: (0, 0)),
            pl.BlockSpec((fine, d_model), lambda i: (0, 0)),
            pl.BlockSpec((fine, d_model), lambda i: (0, 0)),
        ],
        out_specs=pl.BlockSpec((block, d_model), lambda i: (i, 0)),
        out_shape=jax.ShapeDtypeStruct((seq_len, d_model), jnp.float32),
    )(a1, a2, m1, m2, b1, b2)
